# baseline (speedup 1.0000x reference)
import jax
import jax.numpy as jnp
from jax.experimental import pallas as pl
from jax.experimental.pallas import tpu as pltpu

_N = 4096
_BC = 256
_NBLK = _N // _BC
_K0 = _N // 3
_K1 = 2 * (_N // 3)
_MASK = 0x7FFFFFFF


def _kth_key(keys, k):
    n_neg = jnp.sum((keys < jnp.int32(0)).astype(jnp.int32))
    is_neg = jnp.int32(k + 1) <= n_neg
    lo0 = jnp.where(is_neg, jnp.int32(-(2 ** 31)), jnp.int32(0))
    hi0 = jnp.where(is_neg, jnp.int32(-1), jnp.int32(2 ** 31 - 1))

    def body(_, lohi):
        lo, hi = lohi
        mid = lo + (hi - lo) // 2
        cnt = jnp.sum((keys <= mid).astype(jnp.int32))
        ge = cnt >= jnp.int32(k + 1)
        return jnp.where(ge, lo, mid + 1), jnp.where(ge, mid, hi)

    lo, _ = jax.lax.fori_loop(0, 31, body, (lo0, hi0))
    return lo


def _body(scores_full_ref, scores_blk_ref, w_ref, o_ref, thr_ref):
    j = pl.program_id(0)

    @pl.when(j == 0)
    def _():
        s = scores_full_ref[...]
        b = jax.lax.bitcast_convert_type(s, jnp.int32)
        keys = b ^ ((b >> 31) & jnp.int32(_MASK))

        def unmap(kk):
            return jax.lax.bitcast_convert_type(
                jnp.where(kk >= 0, kk, kk ^ jnp.int32(_MASK)), jnp.float32)

        thr_ref[0] = unmap(_kth_key(keys, _K0))
        thr_ref[1] = unmap(_kth_key(keys, _K1))

    t0 = thr_ref[0]
    t1 = thr_ref[1]
    s = scores_blk_ref[...]
    half = jnp.where(s <= t0, 2.0, jnp.where(s <= t1, 8.0, 32.0))
    q_min = -half
    q_max = half - 1.0

    w = w_ref[...]
    mn = jnp.min(w, axis=0, keepdims=True)
    mx = jnp.max(w, axis=0, keepdims=True)
    scale = (mx - mn) / (q_max - q_min)
    scale = jnp.where(jnp.abs(scale) < 1e-6, jnp.float32(1e-6), scale)
    inv = 1.0 / scale
    zp = jnp.clip(jnp.round(q_min - mn / scale), q_min, q_max)
    q = jnp.clip(jnp.round(w * inv) + zp, -128.0, 127.0)
    o_ref[...] = (q - zp) * scale


def kernel(weight, scores):
    scores_full = scores.reshape(32, 128)
    scores_row = scores.reshape(1, _N)
    return pl.pallas_call(
        _body,
        grid=(_NBLK,),
        in_specs=[
            pl.BlockSpec((32, 128), lambda j: (0, 0)),
            pl.BlockSpec((1, _BC), lambda j: (0, j)),
            pl.BlockSpec((_N, _BC), lambda j: (0, j)),
        ],
        out_specs=pl.BlockSpec((_N, _BC), lambda j: (0, j)),
        out_shape=jax.ShapeDtypeStruct((_N, _N), jnp.float32),
        scratch_shapes=[pltpu.SMEM((2,), jnp.float32)],
        compiler_params=pltpu.CompilerParams(
            dimension_semantics=("arbitrary",),
        ),
    )(scores_full, scores_row, weight)


# speedup vs baseline: 1.0261x; 1.0261x over previous
import jax
import jax.numpy as jnp
from jax.experimental import pallas as pl
from jax.experimental.pallas import tpu as pltpu

_N = 4096
_BC = 512
_NBLK = _N // _BC
_K0 = _N // 3
_K1 = 2 * (_N // 3)
_MASK = 0x7FFFFFFF


def _kth_key(keys, k):
    n_neg = jnp.sum((keys < jnp.int32(0)).astype(jnp.int32))
    is_neg = jnp.int32(k + 1) <= n_neg
    lo0 = jnp.where(is_neg, jnp.int32(-(2 ** 31)), jnp.int32(0))
    hi0 = jnp.where(is_neg, jnp.int32(-1), jnp.int32(2 ** 31 - 1))

    def body(_, lohi):
        lo, hi = lohi
        mid = lo + (hi - lo) // 2
        cnt = jnp.sum((keys <= mid).astype(jnp.int32))
        ge = cnt >= jnp.int32(k + 1)
        return jnp.where(ge, lo, mid + 1), jnp.where(ge, mid, hi)

    lo, _ = jax.lax.fori_loop(0, 31, body, (lo0, hi0))
    return lo


def _body(scores_full_ref, scores_blk_ref, w_ref, o_ref, thr_ref):
    j = pl.program_id(0)

    @pl.when(j == 0)
    def _():
        s = scores_full_ref[...]
        b = jax.lax.bitcast_convert_type(s, jnp.int32)
        keys = b ^ ((b >> 31) & jnp.int32(_MASK))

        def unmap(kk):
            return jax.lax.bitcast_convert_type(
                jnp.where(kk >= 0, kk, kk ^ jnp.int32(_MASK)), jnp.float32)

        thr_ref[0] = unmap(_kth_key(keys, _K0))
        thr_ref[1] = unmap(_kth_key(keys, _K1))

    t0 = thr_ref[0]
    t1 = thr_ref[1]
    s = scores_blk_ref[...]
    half = jnp.where(s <= t0, 2.0, jnp.where(s <= t1, 8.0, 32.0))
    q_min = -half
    q_max = half - 1.0

    w = w_ref[...]
    mn = jnp.min(w, axis=0, keepdims=True)
    mx = jnp.max(w, axis=0, keepdims=True)
    scale = (mx - mn) / (q_max - q_min)
    scale = jnp.where(jnp.abs(scale) < 1e-6, jnp.float32(1e-6), scale)
    inv = 1.0 / scale
    zp = jnp.clip(jnp.round(q_min - mn / scale), q_min, q_max)
    q = jnp.clip(jnp.round(w * inv) + zp, -128.0, 127.0)
    o_ref[...] = (q - zp) * scale


def kernel(weight, scores):
    scores_full = scores.reshape(32, 128)
    scores_row = scores.reshape(1, _N)
    return pl.pallas_call(
        _body,
        grid=(_NBLK,),
        in_specs=[
            pl.BlockSpec((32, 128), lambda j: (0, 0)),
            pl.BlockSpec((1, _BC), lambda j: (0, j)),
            pl.BlockSpec((_N, _BC), lambda j: (0, j)),
        ],
        out_specs=pl.BlockSpec((_N, _BC), lambda j: (0, j)),
        out_shape=jax.ShapeDtypeStruct((_N, _N), jnp.float32),
        scratch_shapes=[pltpu.SMEM((2,), jnp.float32)],
        compiler_params=pltpu.CompilerParams(
            dimension_semantics=("arbitrary",),
        ),
    )(scores_full, scores_row, weight)
